# baseline (speedup 1.0000x reference)
import functools

import jax
import jax.numpy as jnp
from jax import lax
from jax.experimental import pallas as pl
from jax.experimental.pallas import tpu as pltpu
from jax.experimental.pallas import tpu_sc as plsc

B, L, V, D = 16384, 200, 81, 64
VP = 81
TP = 88
P = V * VP
T = B * L
NC, NS = 2, 16
NW = NC * NS
BR = B // NW
GR = 128
NG = BR // GR
PC = L // 2
XR = L * GR // 128
XROWS2 = T // 128


def _body(table_hbm, idx_hbm, out_hbm, pair_s, xs_v, pidx_v, wb0, wb1, wb2,
          gsem, wsem0, wsem1, wsem2):
    cid = lax.axis_index("c")
    sid = lax.axis_index("s")
    wid = sid * NC + cid
    xrow_base = wid * (L * BR // 128)
    orow_base = wid * BR

    pltpu.sync_copy(table_hbm, wb1.at[pl.ds(0, TP)])

    def build_i(k, carry):
        i = sid + k * NS

        @pl.when(i < V)
        def _():
            lefts = [wb1[i, pl.ds(c * 16, 16)] for c in range(D // 16)]

            def put(j, carry2):
                for c in range(D // 16):
                    wb0[j, pl.ds(c * 16, 16)] = lefts[c]
                    wb0[j, pl.ds(D + c * 16, 16)] = wb1[j, pl.ds(c * 16, 16)]
                return carry2

            lax.fori_loop(0, V, put, 0)
            pltpu.sync_copy(wb0.at[pl.ds(0, VP)], pair_s.at[pl.ds(i * VP, VP)])
        return carry

    lax.fori_loop(0, (V + NS - 1) // NS, build_i, 0)
    plsc.subcore_barrier()

    wbufs = (wb0, wb1, wb2)
    wsems = (wsem0, wsem1, wsem2)
    iot = lax.iota(jnp.int32, 16)

    def compute_pidx(c, carry):
        for q in range(GR // 16):
            flat_e = (q * 16 + iot) * L + 2 * c
            flat_o = flat_e + 1
            even = plsc.load_gather(
                xs_v, [lax.shift_right_logical(flat_e, 7), flat_e & 127]
            )
            odd = plsc.load_gather(
                xs_v, [lax.shift_right_logical(flat_o, 7), flat_o & 127]
            )
            pidx_v[c & 3, pl.ds(q * 16, 16)] = even * VP + odd
        return carry

    def fire(blk, buf):
        return pltpu.async_copy(
            pair_s.at[pidx_v.at[blk & 3]], wbufs[buf], gsem
        )

    def group(g, carry):
        pltpu.sync_copy(idx_hbm.at[pl.ds(xrow_base + g * XR, XR)], xs_v)
        compute_pidx(0, 0)
        compute_pidx(1, 0)
        compute_pidx(2, 0)
        r0 = orow_base + g * GR

        def out_slice(blk):
            return out_hbm.at[pl.ds(r0, GR), pl.ds(blk * 128, 128)]

        def block3(h, carry2):
            for t in range(3):
                b = 3 * h + t

                @pl.when(h > 0)
                def _():
                    pltpu.make_async_copy(
                        wbufs[t], out_slice(b - 3), wsems[t]
                    ).wait()

                gd = fire(b, t)

                @pl.when(b + 3 < PC)
                def _():
                    compute_pidx(b + 3, 0)

                gd.wait()
                pltpu.async_copy(wbufs[t], out_slice(b), wsems[t])
            return carry2

        lax.fori_loop(0, PC // 3, block3, 0)
        pltpu.make_async_copy(wbufs[0], out_slice(PC - 4), wsems[0]).wait()
        fire(PC - 1, 0).wait()
        pltpu.async_copy(wbufs[0], out_slice(PC - 1), wsems[0])
        pltpu.make_async_copy(wbufs[1], out_slice(PC - 3), wsems[1]).wait()
        pltpu.make_async_copy(wbufs[2], out_slice(PC - 2), wsems[2]).wait()
        pltpu.make_async_copy(wbufs[0], out_slice(PC - 1), wsems[0]).wait()
        return carry

    lax.fori_loop(0, NG, group, 0)


_emb = functools.partial(
    pl.kernel,
    out_type=jax.ShapeDtypeStruct((B, L * D), jnp.float32),
    mesh=plsc.VectorSubcoreMesh(
        core_axis_name="c", subcore_axis_name="s", num_cores=NC, num_subcores=NS
    ),
    scratch_types=[
        pltpu.VMEM_SHARED((P, 128), jnp.float32),
        pltpu.VMEM((XR, 128), jnp.int32),
        pltpu.VMEM((4, GR), jnp.int32),
        pltpu.VMEM((GR, 128), jnp.float32),
        pltpu.VMEM((GR, 128), jnp.float32),
        pltpu.VMEM((GR, 128), jnp.float32),
        pltpu.SemaphoreType.DMA,
        pltpu.SemaphoreType.DMA,
        pltpu.SemaphoreType.DMA,
        pltpu.SemaphoreType.DMA,
    ],
    compiler_params=pltpu.CompilerParams(needs_layout_passes=False),
)(_body)


def kernel(x, table):
    table_p = jnp.pad(table, ((0, TP - V), (0, 128 - D)))
    return _emb(table_p, x.reshape(XROWS2, 128))
